# baseline (speedup 1.0000x reference)
import functools

import jax
import jax.numpy as jnp
from jax import lax
from jax.experimental import pallas as pl
from jax.experimental.pallas import tpu as pltpu

H = 128
N_NODES = {"show": 10000, "performance": 100000, "song": 10000}
EDGE_TYPES = [
    ("show", "has_performance", "performance"),
    ("performance", "of_song", "song"),
    ("song", "transitioned_to", "song"),
    ("show", "setlist_neighbor", "show"),
    ("performance", "rev_has_performance", "show"),
    ("song", "rev_of_song", "performance"),
    ("song", "rev_transitioned_to", "song"),
]
N_EDGES = 100000

DST_GROUPS = {
    "show": ["setlist_neighbor", "rev_has_performance"],
    "performance": ["has_performance", "rev_of_song"],
    "song": ["of_song", "transitioned_to", "rev_transitioned_to"],
}
SRC_OF = {r: s for (s, r, d) in EDGE_TYPES}

BLK = 400


def _dense_body(n_r, relu, *refs):
    idx = 0
    s_refs = []
    c_refs = []
    for _ in range(n_r):
        s_refs.append(refs[idx]); idx += 1
        c_refs.append(refs[idx]); idx += 1
    x_ref = refs[idx]; idx += 1
    wrs_ref = refs[idx]; idx += 1
    bs_ref = refs[idx]; idx += 1
    wl_refs = refs[idx:idx + n_r]; idx += n_r
    out_ref = refs[idx]

    acc = lax.dot_general(
        x_ref[...], wrs_ref[...], (((1,), (0,)), ((), ())),
        preferred_element_type=jnp.float32,
        precision=lax.Precision.HIGHEST,
    ) + bs_ref[...]
    for r in range(n_r):
        cnt = jnp.maximum(c_refs[r][...], 1.0)
        agg = s_refs[r][...] / cnt
        acc = acc + lax.dot_general(
            agg, wl_refs[r][...], (((1,), (0,)), ((), ())),
            preferred_element_type=jnp.float32,
            precision=lax.Precision.HIGHEST,
        )
    if relu:
        acc = jnp.maximum(acc, 0.0)
    out_ref[...] = acc


def _dense_layer(n_r, relu, s_list, cnt_list, x, wrs, bs, wl_list):
    n = x.shape[0]
    grid = (n // BLK,)
    row_spec = pl.BlockSpec((BLK, H), lambda i: (i, 0))
    cnt_spec = pl.BlockSpec((BLK, 1), lambda i: (i, 0))
    full_spec = pl.BlockSpec((H, H), lambda i: (0, 0))
    bias_spec = pl.BlockSpec((1, H), lambda i: (0, 0))
    in_specs = []
    args = []
    for r in range(n_r):
        in_specs += [row_spec, cnt_spec]
        args += [s_list[r], cnt_list[r]]
    in_specs += [row_spec, full_spec, bias_spec] + [full_spec] * n_r
    args += [x, wrs, bs] + list(wl_list)
    return pl.pallas_call(
        functools.partial(_dense_body, n_r, relu),
        grid=grid,
        in_specs=in_specs,
        out_specs=row_spec,
        out_shape=jax.ShapeDtypeStruct((n, H), jnp.float32),
    )(*args)


def _aggregate(xd, eis):
    s_out, cnt_out = {}, {}
    for (s, r, d) in EDGE_TYPES:
        ei = eis[r]
        n_dst = N_NODES[d]
        msg = jnp.take(xd[s], ei[0], axis=0)
        s_out[r] = jax.ops.segment_sum(msg, ei[1], num_segments=n_dst)
        cnt_out[r] = jax.ops.segment_sum(
            jnp.ones((ei.shape[1], 1), jnp.float32), ei[1], num_segments=n_dst)
    return s_out, cnt_out


def _layer(xd, eis, params, layer, relu):
    s_out, cnt_out = _aggregate(xd, eis)
    out = {}
    for d, rels in DST_GROUPS.items():
        wrs = sum(params[f"Wr{layer}_{r}"] for r in rels)
        bs = sum(params[f"b{layer}_{r}"] for r in rels).reshape(1, H)
        out[d] = _dense_layer(
            len(rels), relu,
            [s_out[r] for r in rels],
            [cnt_out[r] for r in rels],
            xd[d], wrs, bs,
            [params[f"Wl{layer}_{r}"] for r in rels],
        )
    return out


def kernel(x_show, x_performance, x_song, ei_has_performance, ei_of_song, ei_transitioned_to, ei_setlist_neighbor, ei_rev_has_performance, ei_rev_of_song, ei_rev_transitioned_to, Wl1_has_performance, Wr1_has_performance, b1_has_performance, Wl1_of_song, Wr1_of_song, b1_of_song, Wl1_transitioned_to, Wr1_transitioned_to, b1_transitioned_to, Wl1_setlist_neighbor, Wr1_setlist_neighbor, b1_setlist_neighbor, Wl1_rev_has_performance, Wr1_rev_has_performance, b1_rev_has_performance, Wl1_rev_of_song, Wr1_rev_of_song, b1_rev_of_song, Wl1_rev_transitioned_to, Wr1_rev_transitioned_to, b1_rev_transitioned_to, Wl2_has_performance, Wr2_has_performance, b2_has_performance, Wl2_of_song, Wr2_of_song, b2_of_song, Wl2_transitioned_to, Wr2_transitioned_to, b2_transitioned_to, Wl2_setlist_neighbor, Wr2_setlist_neighbor, b2_setlist_neighbor, Wl2_rev_has_performance, Wr2_rev_has_performance, b2_rev_has_performance, Wl2_rev_of_song, Wr2_rev_of_song, b2_rev_of_song, Wl2_rev_transitioned_to, Wr2_rev_transitioned_to, b2_rev_transitioned_to):
    kw = dict(locals())
    eis = {r: kw[f"ei_{r}"] for (_, r, _) in EDGE_TYPES}
    params = {k: v for k, v in kw.items()
              if k[:2] in ("Wl", "Wr") or k[0] == "b"}
    xd = {"show": x_show, "performance": x_performance, "song": x_song}
    h = _layer(xd, eis, params, 1, relu=True)
    h = _layer(h, eis, params, 2, relu=False)
    return (h["show"], h["performance"], h["song"])
